# baseline (speedup 1.0000x reference)
import jax
import jax.numpy as jnp
from jax.experimental import pallas as pl
from jax.experimental.pallas import tpu as pltpu

_BT = 2048


def _router_body(x_ref, w_ref, probs_ref, logits_ref):
    x = x_ref[...]
    w = w_ref[...]
    logits = jnp.dot(x, w, preferred_element_type=jnp.float32)
    m = jnp.max(logits, axis=-1, keepdims=True)
    e = jnp.exp(logits - m)
    probs = e / jnp.sum(e, axis=-1, keepdims=True)
    logits_ref[...] = logits
    probs_ref[...] = probs


def kernel(inputs, num_experts, w):
    n_tokens, d = inputs.shape
    n_exp = w.shape[1]
    grid = (n_tokens // _BT,)
    probs, logits = pl.pallas_call(
        _router_body,
        grid=grid,
        in_specs=[
            pl.BlockSpec((_BT, d), lambda i: (i, 0)),
            pl.BlockSpec((d, n_exp), lambda i: (0, 0)),
        ],
        out_specs=[
            pl.BlockSpec((_BT, n_exp), lambda i: (i, 0)),
            pl.BlockSpec((_BT, n_exp), lambda i: (i, 0)),
        ],
        out_shape=[
            jax.ShapeDtypeStruct((n_tokens, n_exp), jnp.float32),
            jax.ShapeDtypeStruct((n_tokens, n_exp), jnp.float32),
        ],
        compiler_params=pltpu.CompilerParams(
            dimension_semantics=("arbitrary",),
        ),
    )(inputs, w)
    return (probs, logits, 0)


# speedup vs baseline: 1.0337x; 1.0337x over previous
import jax
import jax.numpy as jnp
from jax import lax
from jax.experimental import pallas as pl
from jax.experimental.pallas import tpu as pltpu

_CHUNK = 2048
_NBUF = 4


def _router_body(x_hbm, w_ref, probs_hbm, logits_hbm,
                 xbuf, pbuf, lbuf, in_sem, p_sem, l_sem):
    n_tokens = x_hbm.shape[0]
    n_chunks = n_tokens // _CHUNK
    w = w_ref[...]

    def in_copy(chunk, buf):
        return pltpu.make_async_copy(
            x_hbm.at[pl.ds(chunk * _CHUNK, _CHUNK), :],
            xbuf.at[buf],
            in_sem.at[buf],
        )

    for b in range(_NBUF):
        in_copy(b, b).start()

    def step(i, carry):
        buf = lax.rem(i, _NBUF)
        in_copy(i, buf).wait()

        @pl.when(i >= _NBUF)
        def _():
            pltpu.make_async_copy(
                pbuf.at[buf], probs_hbm.at[pl.ds(0, _CHUNK), :], p_sem.at[buf]
            ).wait()
            pltpu.make_async_copy(
                lbuf.at[buf], logits_hbm.at[pl.ds(0, _CHUNK), :], l_sem.at[buf]
            ).wait()

        x = xbuf[buf]
        logits = jnp.dot(x, w, preferred_element_type=jnp.float32)
        m = jnp.max(logits, axis=-1, keepdims=True)
        e = jnp.exp(logits - m)
        probs = e / jnp.sum(e, axis=-1, keepdims=True)
        pbuf[buf] = probs
        lbuf[buf] = logits

        pltpu.make_async_copy(
            pbuf.at[buf], probs_hbm.at[pl.ds(i * _CHUNK, _CHUNK), :], p_sem.at[buf]
        ).start()
        pltpu.make_async_copy(
            lbuf.at[buf], logits_hbm.at[pl.ds(i * _CHUNK, _CHUNK), :], l_sem.at[buf]
        ).start()

        @pl.when(i + _NBUF < n_chunks)
        def _():
            in_copy(i + _NBUF, buf).start()

        return carry

    lax.fori_loop(0, n_chunks, step, 0)

    for b in range(_NBUF):
        pltpu.make_async_copy(
            pbuf.at[b], probs_hbm.at[pl.ds(0, _CHUNK), :], p_sem.at[b]
        ).wait()
        pltpu.make_async_copy(
            lbuf.at[b], logits_hbm.at[pl.ds(0, _CHUNK), :], l_sem.at[b]
        ).wait()


def kernel(inputs, num_experts, w):
    n_tokens, d = inputs.shape
    n_exp = w.shape[1]
    probs, logits = pl.pallas_call(
        _router_body,
        in_specs=[
            pl.BlockSpec(memory_space=pl.ANY),
            pl.BlockSpec(memory_space=pltpu.VMEM),
        ],
        out_specs=[
            pl.BlockSpec(memory_space=pl.ANY),
            pl.BlockSpec(memory_space=pl.ANY),
        ],
        out_shape=[
            jax.ShapeDtypeStruct((n_tokens, n_exp), jnp.float32),
            jax.ShapeDtypeStruct((n_tokens, n_exp), jnp.float32),
        ],
        scratch_shapes=[
            pltpu.VMEM((_NBUF, _CHUNK, d), jnp.float32),
            pltpu.VMEM((_NBUF, _CHUNK, n_exp), jnp.float32),
            pltpu.VMEM((_NBUF, _CHUNK, n_exp), jnp.float32),
            pltpu.SemaphoreType.DMA((_NBUF,)),
            pltpu.SemaphoreType.DMA((_NBUF,)),
            pltpu.SemaphoreType.DMA((_NBUF,)),
        ],
    )(inputs, w)
    return (probs, logits, 0)
